# Initial kernel scaffold; baseline (speedup 1.0000x reference)
import jax
import jax.numpy as jnp
from jax.experimental import pallas as pl


def kernel(user_ids, item_ids, user_emb, item_emb, user_bias, item_bias, global_bias):
    raise NotImplementedError("write your pallas kernel here")


# speedup vs baseline: 1.4912x; 1.4912x over previous
import functools
import jax
import jax.numpy as jnp
from jax import lax
from jax.experimental import pallas as pl
from jax.experimental.pallas import tpu as pltpu
from jax.experimental.pallas import tpu_sc as plsc

_NC = 2
_NS = 16
_NW = _NC * _NS
_B = 16384
_D = 128
_BPW = _B // _NW
_CH = 256
_NCHUNK = _BPW // _CH
_GRP = _CH // 16
_PSTRIDE = 17


def _mf_body(uids, iids, uemb, qemb, ubias, ibias, out,
             uidx_v, iidx_v, urows_v, qrows_v, ub_v, ib_v, out_v, pbuf_v,
             sem_u, sem_q, sem_ub, sem_ib):
    cid = lax.axis_index("c")
    sid = lax.axis_index("s")
    wid = sid * _NC + cid
    base = wid * _BPW
    lanes = lax.iota(jnp.int32, 16)
    lanes_p = lanes * _PSTRIDE

    for chunk in range(_NCHUNK):
        off = base + chunk * _CH
        pltpu.sync_copy(uids.at[pl.ds(off, _CH)], uidx_v)
        pltpu.sync_copy(iids.at[pl.ds(off, _CH)], iidx_v)
        cu = pltpu.async_copy(uemb.at[uidx_v], urows_v, sem_u)
        cq = pltpu.async_copy(qemb.at[iidx_v], qrows_v, sem_q)
        cub = pltpu.async_copy(ubias.at[uidx_v], ub_v, sem_ub)
        cib = pltpu.async_copy(ibias.at[iidx_v], ib_v, sem_ib)
        cu.wait()
        cq.wait()
        cub.wait()
        cib.wait()

        def grp(g, carry):
            rbase = g * 16
            for r in range(16):
                row = rbase + r
                p = urows_v[row, pl.ds(0, 16)] * qrows_v[row, pl.ds(0, 16)]
                for k in range(1, 8):
                    p = p + (urows_v[row, pl.ds(k * 16, 16)]
                             * qrows_v[row, pl.ds(k * 16, 16)])
                pbuf_v[pl.ds(r * _PSTRIDE, 16)] = p
            acc = plsc.load_gather(pbuf_v, [lanes_p])
            for c in range(1, 16):
                acc = acc + plsc.load_gather(pbuf_v, [lanes_p + c])
            res = acc + ub_v[pl.ds(rbase, 16)] + ib_v[pl.ds(rbase, 16)]
            out_v[pl.ds(rbase, 16)] = res
            return carry

        lax.fori_loop(0, _GRP, grp, 0)
        pltpu.sync_copy(out_v, out.at[pl.ds(off, _CH)])


@functools.partial(
    pl.kernel,
    out_type=jax.ShapeDtypeStruct((_B,), jnp.float32),
    mesh=plsc.VectorSubcoreMesh(core_axis_name="c", subcore_axis_name="s"),
    compiler_params=pltpu.CompilerParams(needs_layout_passes=False),
    scratch_types=[
        pltpu.VMEM((_CH,), jnp.int32),
        pltpu.VMEM((_CH,), jnp.int32),
        pltpu.VMEM((_CH, _D), jnp.float32),
        pltpu.VMEM((_CH, _D), jnp.float32),
        pltpu.VMEM((_CH,), jnp.float32),
        pltpu.VMEM((_CH,), jnp.float32),
        pltpu.VMEM((_CH,), jnp.float32),
        pltpu.VMEM((16 * _PSTRIDE,), jnp.float32),
        pltpu.SemaphoreType.DMA,
        pltpu.SemaphoreType.DMA,
        pltpu.SemaphoreType.DMA,
        pltpu.SemaphoreType.DMA,
    ],
)
def _mf_kernel(*refs):
    _mf_body(*refs)


def kernel(user_ids, item_ids, user_emb, item_emb, user_bias, item_bias,
           global_bias):
    out = _mf_kernel(user_ids, item_ids, user_emb, item_emb,
                     user_bias.reshape(-1), item_bias.reshape(-1))
    return out + global_bias[0]


# speedup vs baseline: 1.5704x; 1.0531x over previous
import functools
import jax
import jax.numpy as jnp
from jax import lax
from jax.experimental import pallas as pl
from jax.experimental.pallas import tpu as pltpu
from jax.experimental.pallas import tpu_sc as plsc

_NC = 2
_NS = 16
_NW = _NC * _NS
_B = 16384
_D = 128
_BPW = _B // _NW
_CH = 64
_NCHUNK = _BPW // _CH
_NBUF = 2
_GRP = _CH // 16
_PSTRIDE = 17


def _mf_body(uids, iids, uemb, qemb, ubias, ibias, gbias, out,
             uidx_v, iidx_v, ub_v, ib_v, out_v, pbuf_v,
             urows0, urows1, qrows0, qrows1,
             gb_v, sem0, sem1, sem_b):
    urows = (urows0, urows1)
    qrows = (qrows0, qrows1)
    sems = (sem0, sem1)

    cid = lax.axis_index("c")
    sid = lax.axis_index("s")
    wid = sid * _NC + cid
    base = wid * _BPW
    lanes = lax.iota(jnp.int32, 16)
    lanes_p = lanes * _PSTRIDE
    zeros16 = jnp.zeros((16,), jnp.int32)

    pltpu.sync_copy(gbias, gb_v)
    pltpu.sync_copy(uids.at[pl.ds(base, _BPW)], uidx_v)
    pltpu.sync_copy(iids.at[pl.ds(base, _BPW)], iidx_v)
    cub = pltpu.async_copy(ubias.at[uidx_v], ub_v, sem_b)
    cib = pltpu.async_copy(ibias.at[iidx_v], ib_v, sem_b)

    def issue(c):
        b = c % _NBUF
        cu = pltpu.async_copy(
            uemb.at[uidx_v.at[pl.ds(c * _CH, _CH)]], urows[b], sems[b])
        cq = pltpu.async_copy(
            qemb.at[iidx_v.at[pl.ds(c * _CH, _CH)]], qrows[b], sems[b])
        return cu, cq

    inflight = [issue(0), issue(1)]
    cub.wait()
    cib.wait()
    mu = plsc.load_gather(gb_v, [zeros16])

    for c in range(_NCHUNK):
        b = c % _NBUF
        cu, cq = inflight[c]
        cu.wait()
        cq.wait()
        ur = urows[b]
        qr = qrows[b]

        def grp(g, carry):
            rbase = g * 16
            for r in range(16):
                row = rbase + r
                p = ur[row, pl.ds(0, 16)] * qr[row, pl.ds(0, 16)]
                for k in range(1, 8):
                    p = p + (ur[row, pl.ds(k * 16, 16)]
                             * qr[row, pl.ds(k * 16, 16)])
                pbuf_v[pl.ds(r * _PSTRIDE, 16)] = p
            acc = plsc.load_gather(pbuf_v, [lanes_p])
            for col in range(1, 16):
                acc = acc + plsc.load_gather(pbuf_v, [lanes_p + col])
            obase = c * _CH + rbase
            res = (acc + ub_v[pl.ds(obase, 16)]
                   + ib_v[pl.ds(obase, 16)] + mu)
            out_v[pl.ds(obase, 16)] = res
            return carry

        lax.fori_loop(0, _GRP, grp, 0)
        if c + _NBUF < _NCHUNK:
            inflight.append(issue(c + _NBUF))

    pltpu.sync_copy(out_v, out.at[pl.ds(base, _BPW)])


@functools.partial(
    pl.kernel,
    out_type=jax.ShapeDtypeStruct((_B,), jnp.float32),
    mesh=plsc.VectorSubcoreMesh(core_axis_name="c", subcore_axis_name="s"),
    compiler_params=pltpu.CompilerParams(needs_layout_passes=False),
    scratch_types=[
        pltpu.VMEM((_BPW,), jnp.int32),
        pltpu.VMEM((_BPW,), jnp.int32),
        pltpu.VMEM((_BPW,), jnp.float32),
        pltpu.VMEM((_BPW,), jnp.float32),
        pltpu.VMEM((_BPW,), jnp.float32),
        pltpu.VMEM((16 * _PSTRIDE,), jnp.float32),
        pltpu.VMEM((_CH, _D), jnp.float32),
        pltpu.VMEM((_CH, _D), jnp.float32),
        pltpu.VMEM((_CH, _D), jnp.float32),
        pltpu.VMEM((_CH, _D), jnp.float32),
        pltpu.VMEM((1,), jnp.float32),
        pltpu.SemaphoreType.DMA,
        pltpu.SemaphoreType.DMA,
        pltpu.SemaphoreType.DMA,
    ],
)
def _mf_kernel(*refs):
    _mf_body(*refs)


def kernel(user_ids, item_ids, user_emb, item_emb, user_bias, item_bias,
           global_bias):
    return _mf_kernel(user_ids, item_ids, user_emb, item_emb,
                      user_bias.reshape(-1), item_bias.reshape(-1),
                      global_bias)


# speedup vs baseline: 1.6975x; 1.0809x over previous
import functools
import jax
import jax.numpy as jnp
from jax import lax
from jax.experimental import pallas as pl
from jax.experimental.pallas import tpu as pltpu
from jax.experimental.pallas import tpu_sc as plsc

_NC = 2
_NS = 16
_NW = _NC * _NS
_B = 16384
_D = 128
_BPW = _B // _NW
_CH = 64
_NCHUNK = _BPW // _CH
_NPAIR = _NCHUNK // 2
_GRP = _CH // 16
_PSTRIDE = 17


def _mf_body(uids, iids, uemb, qemb, ubias, ibias, gbias, out,
             uidx_v, iidx_v, ub_v, ib_v, out_v, pbuf_v,
             urows0, urows1, qrows0, qrows1,
             gb_v, sem0, sem1, sem_b):
    urows = (urows0, urows1)
    qrows = (qrows0, qrows1)
    sems = (sem0, sem1)

    cid = lax.axis_index("c")
    sid = lax.axis_index("s")
    wid = sid * _NC + cid
    base = wid * _BPW
    lanes = lax.iota(jnp.int32, 16)
    lanes_p = lanes * _PSTRIDE
    zeros16 = jnp.zeros((16,), jnp.int32)

    pltpu.sync_copy(gbias, gb_v)
    pltpu.sync_copy(uids.at[pl.ds(base, _BPW)], uidx_v)
    pltpu.sync_copy(iids.at[pl.ds(base, _BPW)], iidx_v)
    cub = pltpu.async_copy(ubias.at[uidx_v], ub_v, sem_b)
    cib = pltpu.async_copy(ibias.at[iidx_v], ib_v, sem_b)

    def issue(c, b):
        pltpu.async_copy(
            uemb.at[uidx_v.at[pl.ds(c * _CH, _CH)]], urows[b], sems[b])
        pltpu.async_copy(
            qemb.at[iidx_v.at[pl.ds(c * _CH, _CH)]], qrows[b], sems[b])

    def drain(b):
        pltpu.make_async_copy(
            uemb.at[uidx_v.at[pl.ds(0, _CH)]], urows[b], sems[b]).wait()
        pltpu.make_async_copy(
            qemb.at[iidx_v.at[pl.ds(0, _CH)]], qrows[b], sems[b]).wait()

    issue(0, 0)
    issue(1, 1)
    cub.wait()
    cib.wait()
    mu = plsc.load_gather(gb_v, [zeros16])

    def compute_chunk(cdyn, b):
        ur = urows[b]
        qr = qrows[b]

        def grp(g, carry):
            rbase = g * 16
            for r in range(16):
                row = rbase + r
                p = ur[row, pl.ds(0, 16)] * qr[row, pl.ds(0, 16)]
                for k in range(1, 8):
                    p = p + (ur[row, pl.ds(k * 16, 16)]
                             * qr[row, pl.ds(k * 16, 16)])
                pbuf_v[pl.ds(r * _PSTRIDE, 16)] = p
            acc = plsc.load_gather(pbuf_v, [lanes_p])
            for col in range(1, 16):
                acc = acc + plsc.load_gather(pbuf_v, [lanes_p + col])
            obase = cdyn * _CH + rbase
            res = (acc + ub_v[pl.ds(obase, 16)]
                   + ib_v[pl.ds(obase, 16)] + mu)
            out_v[pl.ds(obase, 16)] = res
            return carry

        lax.fori_loop(0, _GRP, grp, 0)

    def pair(i, carry):
        drain(0)
        compute_chunk(2 * i, 0)

        @pl.when(i < _NPAIR - 1)
        def _():
            issue(2 * i + 2, 0)

        drain(1)
        compute_chunk(2 * i + 1, 1)

        @pl.when(i < _NPAIR - 1)
        def _():
            issue(2 * i + 3, 1)

        return carry

    lax.fori_loop(0, _NPAIR, pair, 0)
    pltpu.sync_copy(out_v, out.at[pl.ds(base, _BPW)])


@functools.partial(
    pl.kernel,
    out_type=jax.ShapeDtypeStruct((_B,), jnp.float32),
    mesh=plsc.VectorSubcoreMesh(core_axis_name="c", subcore_axis_name="s"),
    compiler_params=pltpu.CompilerParams(needs_layout_passes=False),
    scratch_types=[
        pltpu.VMEM((_BPW,), jnp.int32),
        pltpu.VMEM((_BPW,), jnp.int32),
        pltpu.VMEM((_BPW,), jnp.float32),
        pltpu.VMEM((_BPW,), jnp.float32),
        pltpu.VMEM((_BPW,), jnp.float32),
        pltpu.VMEM((16 * _PSTRIDE,), jnp.float32),
        pltpu.VMEM((_CH, _D), jnp.float32),
        pltpu.VMEM((_CH, _D), jnp.float32),
        pltpu.VMEM((_CH, _D), jnp.float32),
        pltpu.VMEM((_CH, _D), jnp.float32),
        pltpu.VMEM((1,), jnp.float32),
        pltpu.SemaphoreType.DMA,
        pltpu.SemaphoreType.DMA,
        pltpu.SemaphoreType.DMA,
    ],
)
def _mf_kernel(*refs):
    _mf_body(*refs)


def kernel(user_ids, item_ids, user_emb, item_emb, user_bias, item_bias,
           global_bias):
    return _mf_kernel(user_ids, item_ids, user_emb, item_emb,
                      user_bias.reshape(-1), item_bias.reshape(-1),
                      global_bias)
